# baseline (speedup 1.0000x reference)
import jax
import jax.numpy as jnp
from jax.experimental import pallas as pl
from jax.experimental.pallas import tpu as pltpu


def _make_kernel(steps_per_e, total):
    def _fused_mlp_kernel(x_ref, w1_ref, b1_ref, w2_ref, b2_ref, out_ref,
                          w1b_s, w2p_s, h_s):
        g = pl.program_id(0)

        @pl.when(jnp.logical_and(g % steps_per_e == 0, g < total))
        def _prep_w1():
            w1b_s[...] = w1_ref[0].astype(jnp.bfloat16)

        @pl.when(g % steps_per_e == 1)
        def _prep_w2():
            bits = pltpu.bitcast(w2_ref[0], jnp.int32)
            b16 = (bits + 0x7FFF + ((bits >> 16) & 1)) >> 16
            w2p_s[...] = pltpu.bitcast((b16 & 0xFFFF).astype(jnp.int32),
                                       jnp.bfloat16)

        hf = h_s[...].astype(jnp.float32)
        hr = jnp.roll(hf, shift=-1, axis=1)
        s = (hf * jax.nn.sigmoid(hf)) * hr
        y = jnp.dot(s.astype(jnp.bfloat16), w2p_s[...],
                    preferred_element_type=jnp.float32) + b2_ref[0]
        out_ref[0, 0] = y

        x = x_ref[0, 0].astype(jnp.bfloat16)
        h = jnp.dot(x, w1b_s[...],
                    preferred_element_type=jnp.float32) + b1_ref[0]
        h_s[...] = h.astype(jnp.bfloat16)

    return _fused_mlp_kernel


def kernel(dispatched, sparsity_remap, w1, w1_bias, w2, w2_bias):
    A, B, E, M, K = dispatched.shape
    N2 = w1.shape[2]
    inter = N2 // 2
    H = w2.shape[2]

    b1 = w1_bias.reshape(E, 1, N2)
    b2 = w2_bias.reshape(E, 1, H)
    x = dispatched.reshape(A * B, E, M, K)

    TM = min(512, M)
    mt = M // TM
    spe = A * B * mt
    total = E * spe
    last = total - 1

    def cur(g):
        return jnp.minimum(g, last)

    def prev(g):
        return jnp.maximum(g - 1, 0)

    out = pl.pallas_call(
        _make_kernel(spe, total),
        grid=(total + 1,),
        in_specs=[
            pl.BlockSpec((1, 1, TM, K),
                         lambda g: ((cur(g) // mt) % (A * B), cur(g) // spe,
                                    cur(g) % mt, 0)),
            pl.BlockSpec((1, K, N2), lambda g: (cur(g) // spe, 0, 0)),
            pl.BlockSpec((1, 1, N2), lambda g: (cur(g) // spe, 0, 0)),
            pl.BlockSpec((1, inter, H), lambda g: (prev(g) // spe, 0, 0)),
            pl.BlockSpec((1, 1, H), lambda g: (prev(g) // spe, 0, 0)),
        ],
        out_specs=pl.BlockSpec(
            (1, 1, TM, H),
            lambda g: (prev(g) // spe, 0, prev(g) % spe, 0)),
        out_shape=jax.ShapeDtypeStruct((E, 1, A * B * M, H), jnp.float32),
        scratch_shapes=[pltpu.VMEM((K, N2), jnp.bfloat16),
                        pltpu.VMEM((N2, H), jnp.bfloat16),
                        pltpu.VMEM((TM, N2), jnp.bfloat16)],
    )(x, w1, b1, w2, b2)

    return out


# speedup vs baseline: 1.1821x; 1.1821x over previous
import jax
import jax.numpy as jnp
from jax.experimental import pallas as pl
from jax.experimental.pallas import tpu as pltpu


def _fused_mlp_kernel(x_ref, w1_ref, b1_ref, w2_ref, b2_ref, out_ref, w2p_s):
    a = pl.program_id(1)
    m = pl.program_id(2)

    @pl.when(jnp.logical_and(a == 0, m == 0))
    def _prep_weights():
        bits = pltpu.bitcast(w2_ref[0], jnp.int32)
        b16 = (bits + 0x7FFF + ((bits >> 16) & 1)) >> 16
        w2p_s[...] = pltpu.bitcast((b16 & 0xFFFF).astype(jnp.int32),
                                   jnp.bfloat16)

    x = x_ref[0, 0].astype(jnp.bfloat16)
    h = jnp.dot(x, w1_ref[0].astype(jnp.bfloat16),
                preferred_element_type=jnp.float32).astype(jnp.bfloat16)
    h = h + b1_ref[0]
    hr = jnp.roll(h, shift=-1, axis=1)
    s = (h * jax.nn.sigmoid(h)) * hr
    y = jnp.dot(s, w2p_s[...],
                preferred_element_type=jnp.float32) + b2_ref[0]
    out_ref[0, 0] = y


def kernel(dispatched, sparsity_remap, w1, w1_bias, w2, w2_bias):
    A, B, E, M, K = dispatched.shape
    N2 = w1.shape[2]
    inter = N2 // 2
    H = w2.shape[2]

    b1 = w1_bias.astype(jnp.bfloat16).reshape(E, 1, N2)
    b2 = w2_bias.reshape(E, 1, H)
    x = dispatched.reshape(A * B, E, M, K)

    TM = min(1024, M)
    mt = M // TM
    grid = (E, A * B, mt)

    out = pl.pallas_call(
        _fused_mlp_kernel,
        grid=grid,
        in_specs=[
            pl.BlockSpec((1, 1, TM, K), lambda e, a, m: (a, e, m, 0)),
            pl.BlockSpec((1, K, N2), lambda e, a, m: (e, 0, 0)),
            pl.BlockSpec((1, 1, N2), lambda e, a, m: (e, 0, 0)),
            pl.BlockSpec((1, inter, H), lambda e, a, m: (e, 0, 0)),
            pl.BlockSpec((1, 1, H), lambda e, a, m: (e, 0, 0)),
        ],
        out_specs=pl.BlockSpec((1, 1, TM, H),
                               lambda e, a, m: (e, 0, a * mt + m, 0)),
        out_shape=jax.ShapeDtypeStruct((E, 1, A * B * M, H), jnp.float32),
        scratch_shapes=[pltpu.VMEM((N2, H), jnp.bfloat16)],
    )(x, w1, b1, w2, b2)

    return out
